# baseline (speedup 1.0000x reference)
import functools

import jax
import jax.numpy as jnp
from jax import lax
from jax.experimental import pallas as pl
from jax.experimental.pallas import tpu as pltpu
from jax.experimental.pallas import tpu_sc as plsc

_CH = 128
_NC = 2
_NS = 16
_NTILE = _NC * _NS


def _leaky(v):
    return jnp.where(v >= 0, v, 0.01 * v)


def _input_transform(x, wcat, t, b, NT, H):
    N, D = x.shape
    YW = NT * H

    def body(x_ref, w_ref, t_ref, b_ref, o_ref):
        y = jnp.dot(x_ref[...], w_ref[...], preferred_element_type=jnp.float32)
        cc = lax.broadcasted_iota(jnp.int32, (N, YW), 1) // H
        sel = jnp.where(cc == t_ref[...], y, 0.0)
        ri = lax.broadcasted_iota(jnp.int32, (YW, H), 0) % H
        ci = lax.broadcasted_iota(jnp.int32, (YW, H), 1)
        S = (ri == ci).astype(jnp.float32)
        h = jnp.dot(sel, S, preferred_element_type=jnp.float32)
        o_ref[...] = _leaky(h + b_ref[...])

    return pl.pallas_call(
        body,
        out_shape=jax.ShapeDtypeStruct((N, H), jnp.float32),
    )(x, wcat, t, b.reshape(1, H))


def _make_mp_round(N, H, KPT, NPAD):
    mesh = plsc.VectorSubcoreMesh(core_axis_name="c", subcore_axis_name="s")
    rpt = NPAD // _NS

    NB = 8
    AHEAD = 4
    assert KPT % NB == 0

    @functools.partial(
        pl.kernel,
        out_type=jax.ShapeDtypeStruct((_NC * NPAD, H), jnp.float32),
        mesh=mesh,
        scratch_types=(
            [pltpu.VMEM((KPT, _CH), jnp.int32)] * 2
            + [pltpu.VMEM((_CH, H), jnp.float32)] * NB
            + [pltpu.VMEM_SHARED((NPAD, H), jnp.float32)]
            + [pltpu.SemaphoreType.DMA] * (2 * NB)
        ),
        compiler_params=pltpu.CompilerParams(use_tc_tiling_on_sc=False),
    )
    def mp(h_hbm, src_hbm, dst_hbm, zero_hbm, out_hbm, src_v, dst_v, *rest):
        rows = rest[:NB]
        agg = rest[NB]
        gsem = rest[NB + 1:2 * NB + 1]
        ssem = rest[2 * NB + 1:]
        c = lax.axis_index("c")
        s = lax.axis_index("s")
        wid = c * _NS + s
        pltpu.sync_copy(zero_hbm.at[pl.ds(s * rpt, rpt)],
                        agg.at[pl.ds(s * rpt, rpt)])
        pltpu.sync_copy(src_hbm.at[pl.ds(wid * KPT, KPT)], src_v)
        pltpu.sync_copy(dst_hbm.at[pl.ds(wid * KPT, KPT)], dst_v)
        plsc.subcore_barrier()

        for u in range(AHEAD):
            pltpu.async_copy(h_hbm.at[src_v.at[u]], rows[u], gsem[u])

        def outer(o, carry):
            base = NB * o
            for u in range(NB):
                i = base + u
                v = (u + AHEAD) % NB
                pltpu.make_async_copy(
                    h_hbm.at[src_v.at[i]], rows[u], gsem[u]).wait()
                pltpu.async_copy(
                    rows[u], agg.at[dst_v.at[i]], ssem[u], add=True)

                @pl.when(i - AHEAD >= 0)
                def _():
                    pltpu.make_async_copy(
                        rows[v], agg.at[dst_v.at[i - AHEAD]], ssem[v]).wait()

                @pl.when(i + AHEAD < KPT)
                def _():
                    pltpu.async_copy(
                        h_hbm.at[src_v.at[i + AHEAD]], rows[v], gsem[v])
            return carry

        lax.fori_loop(0, KPT // NB, outer, 0)
        for k in range(AHEAD):
            i = KPT - AHEAD + k
            u = i % NB
            pltpu.make_async_copy(
                rows[u], agg.at[dst_v.at[i]], ssem[u]).wait()
        plsc.subcore_barrier()
        pltpu.sync_copy(agg.at[pl.ds(s * rpt, rpt)],
                        out_hbm.at[pl.ds(c * NPAD + s * rpt, rpt)])

    return mp


def _mid(p, w, N, NPAD, H):
    def body(p_ref, w_ref, o_ref):
        v = p_ref[:NPAD, :] + p_ref[NPAD:, :]
        o_ref[...] = _leaky(
            jnp.dot(v, w_ref[...], preferred_element_type=jnp.float32))

    return pl.pallas_call(
        body,
        out_shape=jax.ShapeDtypeStruct((NPAD, H), jnp.float32),
    )(p, w)


def _head(q, wh, bh, wo, bo, wlog_t, blog, N, NPAD, H, OUT):
    def body(q_ref, wh_ref, bh_ref, wo_ref, bo_ref, wl_ref, bl_ref,
             out_ref, emb_ref):
        v = q_ref[:NPAD, :] + q_ref[NPAD:, :]
        h2 = _leaky(
            jnp.dot(v, wh_ref[...], preferred_element_type=jnp.float32)
            + bh_ref[...])
        hn = jnp.dot(h2[:N, :], wo_ref[...],
                     preferred_element_type=jnp.float32)
        g = jnp.sum(hn, axis=0, keepdims=True) / N + bo_ref[...]
        emb = _leaky(g)
        logit = jnp.sum(emb * wl_ref[...], axis=1, keepdims=True) + bl_ref[...]
        out_ref[...] = jax.nn.sigmoid(logit)
        emb_ref[...] = emb

    return pl.pallas_call(
        body,
        out_shape=(jax.ShapeDtypeStruct((1, 1), jnp.float32),
                   jax.ShapeDtypeStruct((1, OUT), jnp.float32)),
    )(q, wh, bh.reshape(1, H), wo, bo.reshape(1, OUT),
      wlog_t, blog.reshape(1, 1))


def kernel(x, edge_index, node_types, edge_types, W_in, b_in, W_rel,
           W_hid, b_hid, W_out, b_out, W_log, b_log):
    N, D = x.shape
    NT, _, H = W_in.shape
    OUT = W_out.shape[1]
    E = edge_index.shape[1]

    wcat = jnp.transpose(W_in, (1, 0, 2)).reshape(D, NT * H)
    h = _input_transform(x, wcat, node_types.reshape(N, 1), b_in, NT, H)

    nchunk = -(-E // _CH)
    kpt = -(-nchunk // _NTILE)
    kpt = -(-kpt // 8) * 8
    epad = kpt * _NTILE * _CH
    src = jnp.concatenate(
        [edge_index[0], jnp.zeros((epad - E,), jnp.int32)]).reshape(-1, _CH)
    dst = jnp.concatenate(
        [edge_index[1], jnp.full((epad - E,), N, jnp.int32)]).reshape(-1, _CH)
    npad = -(-N // (_NS * 8)) * (_NS * 8)
    zeros = jnp.zeros((npad, H), jnp.float32)

    mp = _make_mp_round(N, H, kpt, npad)
    p = mp(h, src, dst, zeros)
    h1 = _mid(p, W_rel[0], N, npad, H)
    q = mp(h1, src, dst, zeros)
    out, emb = _head(q, W_hid, b_hid, W_out, b_out,
                     jnp.transpose(W_log), b_log, N, npad, H, OUT)
    return out, emb.reshape(OUT)


# speedup vs baseline: 1.5256x; 1.5256x over previous
import functools

import jax
import jax.numpy as jnp
from jax import lax
from jax.experimental import pallas as pl
from jax.experimental.pallas import tpu as pltpu
from jax.experimental.pallas import tpu_sc as plsc

_CH = 128
_NC = 2
_NS = 16
_NTILE = _NC * _NS


def _leaky(v):
    return jnp.where(v >= 0, v, 0.01 * v)


def _input_transform(x, wcat, t, b, NT, H):
    N, D = x.shape
    YW = NT * H

    def body(x_ref, w_ref, t_ref, b_ref, o_ref):
        y = jnp.dot(x_ref[...], w_ref[...], preferred_element_type=jnp.float32)
        cc = lax.broadcasted_iota(jnp.int32, (N, YW), 1) // H
        sel = jnp.where(cc == t_ref[...], y, 0.0)
        ri = lax.broadcasted_iota(jnp.int32, (YW, H), 0) % H
        ci = lax.broadcasted_iota(jnp.int32, (YW, H), 1)
        S = (ri == ci).astype(jnp.float32)
        h = jnp.dot(sel, S, preferred_element_type=jnp.float32)
        o_ref[...] = _leaky(h + b_ref[...])

    return pl.pallas_call(
        body,
        out_shape=jax.ShapeDtypeStruct((N, H), jnp.float32),
    )(x, wcat, t, b.reshape(1, H))


def _make_mp_round(N, H, KPT, NPAD):
    mesh = plsc.VectorSubcoreMesh(core_axis_name="c", subcore_axis_name="s")
    rpt = NPAD // _NS

    NB = 8
    AHEAD = 4
    assert KPT % NB == 0

    @functools.partial(
        pl.kernel,
        out_type=jax.ShapeDtypeStruct((_NC * NPAD, H), jnp.float32),
        mesh=mesh,
        scratch_types=(
            [pltpu.VMEM((KPT, _CH), jnp.int32)] * 2
            + [pltpu.VMEM((_CH, H), jnp.float32)] * NB
            + [pltpu.VMEM_SHARED((NPAD, H), jnp.float32)]
            + [pltpu.VMEM_SHARED((NPAD, H), jnp.float32)]
            + [pltpu.SemaphoreType.DMA] * (2 * NB)
        ),
        compiler_params=pltpu.CompilerParams(use_tc_tiling_on_sc=False),
    )
    def mp(h_hbm, src_hbm, dst_hbm, zero_hbm, out_hbm, src_v, dst_v, *rest):
        rows = rest[:NB]
        agg = rest[NB]
        hsh = rest[NB + 1]
        gsem = rest[NB + 2:2 * NB + 2]
        ssem = rest[2 * NB + 2:]
        c = lax.axis_index("c")
        s = lax.axis_index("s")
        wid = c * _NS + s
        pltpu.sync_copy(h_hbm.at[pl.ds(s * rpt, rpt)],
                        hsh.at[pl.ds(s * rpt, rpt)])
        pltpu.sync_copy(zero_hbm.at[pl.ds(s * rpt, rpt)],
                        agg.at[pl.ds(s * rpt, rpt)])
        pltpu.sync_copy(src_hbm.at[pl.ds(wid * KPT, KPT)], src_v)
        pltpu.sync_copy(dst_hbm.at[pl.ds(wid * KPT, KPT)], dst_v)
        plsc.subcore_barrier()

        for u in range(AHEAD):
            pltpu.async_copy(hsh.at[src_v.at[u]], rows[u], gsem[u])

        def outer(o, carry):
            base = NB * o
            for u in range(NB):
                i = base + u
                v = (u + AHEAD) % NB
                pltpu.make_async_copy(
                    hsh.at[src_v.at[i]], rows[u], gsem[u]).wait()
                pltpu.async_copy(
                    rows[u], agg.at[dst_v.at[i]], ssem[u], add=True)

                @pl.when(i - AHEAD >= 0)
                def _():
                    pltpu.make_async_copy(
                        rows[v], agg.at[dst_v.at[i - AHEAD]], ssem[v]).wait()

                @pl.when(i + AHEAD < KPT)
                def _():
                    pltpu.async_copy(
                        hsh.at[src_v.at[i + AHEAD]], rows[v], gsem[v])
            return carry

        lax.fori_loop(0, KPT // NB, outer, 0)
        for k in range(AHEAD):
            i = KPT - AHEAD + k
            u = i % NB
            pltpu.make_async_copy(
                rows[u], agg.at[dst_v.at[i]], ssem[u]).wait()
        plsc.subcore_barrier()
        pltpu.sync_copy(agg.at[pl.ds(s * rpt, rpt)],
                        out_hbm.at[pl.ds(c * NPAD + s * rpt, rpt)])

    return mp


def _mid(p, w, N, NPAD, H):
    def body(p_ref, w_ref, o_ref):
        v = p_ref[:NPAD, :] + p_ref[NPAD:, :]
        o_ref[...] = _leaky(
            jnp.dot(v, w_ref[...], preferred_element_type=jnp.float32))

    return pl.pallas_call(
        body,
        out_shape=jax.ShapeDtypeStruct((NPAD, H), jnp.float32),
    )(p, w)


def _head(q, wh, bh, wo, bo, wlog_t, blog, N, NPAD, H, OUT):
    def body(q_ref, wh_ref, bh_ref, wo_ref, bo_ref, wl_ref, bl_ref,
             out_ref, emb_ref):
        v = q_ref[:NPAD, :] + q_ref[NPAD:, :]
        h2 = _leaky(
            jnp.dot(v, wh_ref[...], preferred_element_type=jnp.float32)
            + bh_ref[...])
        hn = jnp.dot(h2[:N, :], wo_ref[...],
                     preferred_element_type=jnp.float32)
        g = jnp.sum(hn, axis=0, keepdims=True) / N + bo_ref[...]
        emb = _leaky(g)
        logit = jnp.sum(emb * wl_ref[...], axis=1, keepdims=True) + bl_ref[...]
        out_ref[...] = jax.nn.sigmoid(logit)
        emb_ref[...] = emb

    return pl.pallas_call(
        body,
        out_shape=(jax.ShapeDtypeStruct((1, 1), jnp.float32),
                   jax.ShapeDtypeStruct((1, OUT), jnp.float32)),
    )(q, wh, bh.reshape(1, H), wo, bo.reshape(1, OUT),
      wlog_t, blog.reshape(1, 1))


def kernel(x, edge_index, node_types, edge_types, W_in, b_in, W_rel,
           W_hid, b_hid, W_out, b_out, W_log, b_log):
    N, D = x.shape
    NT, _, H = W_in.shape
    OUT = W_out.shape[1]
    E = edge_index.shape[1]

    wcat = jnp.transpose(W_in, (1, 0, 2)).reshape(D, NT * H)
    h = _input_transform(x, wcat, node_types.reshape(N, 1), b_in, NT, H)

    nchunk = -(-E // _CH)
    kpt = -(-nchunk // _NTILE)
    kpt = -(-kpt // 8) * 8
    epad = kpt * _NTILE * _CH
    src = jnp.concatenate(
        [edge_index[0], jnp.zeros((epad - E,), jnp.int32)]).reshape(-1, _CH)
    dst = jnp.concatenate(
        [edge_index[1], jnp.full((epad - E,), N, jnp.int32)]).reshape(-1, _CH)
    npad = -(-N // (_NS * 8)) * (_NS * 8)
    zeros = jnp.zeros((npad, H), jnp.float32)
    h = jnp.concatenate([h, jnp.zeros((npad - N, H), jnp.float32)])

    mp = _make_mp_round(N, H, kpt, npad)
    p = mp(h, src, dst, zeros)
    h1 = _mid(p, W_rel[0], N, npad, H)
    q = mp(h1, src, dst, zeros)
    out, emb = _head(q, W_hid, b_hid, W_out, b_out,
                     jnp.transpose(W_log), b_log, N, npad, H, OUT)
    return out, emb.reshape(OUT)


# speedup vs baseline: 1.6785x; 1.1002x over previous
import functools

import jax
import jax.numpy as jnp
from jax import lax
from jax.experimental import pallas as pl
from jax.experimental.pallas import tpu as pltpu
from jax.experimental.pallas import tpu_sc as plsc

_CH = 128
_NC = 2
_NS = 16
_NTILE = _NC * _NS


def _leaky(v):
    return jnp.where(v >= 0, v, 0.01 * v)


def _input_transform(x, wcat, t, b, wrel, NT, H):
    N, D = x.shape
    YW = NT * H

    def body(x_ref, w_ref, t_ref, b_ref, r_ref, o_ref):
        y = jnp.dot(x_ref[...], w_ref[...], preferred_element_type=jnp.float32)
        cc = lax.broadcasted_iota(jnp.int32, (N, YW), 1) // H
        sel = jnp.where(cc == t_ref[...], y, 0.0)
        ri = lax.broadcasted_iota(jnp.int32, (YW, H), 0) % H
        ci = lax.broadcasted_iota(jnp.int32, (YW, H), 1)
        S = (ri == ci).astype(jnp.float32)
        h = jnp.dot(sel, S, preferred_element_type=jnp.float32)
        hr = _leaky(h + b_ref[...])
        o_ref[...] = jnp.dot(hr, r_ref[...], preferred_element_type=jnp.float32)

    return pl.pallas_call(
        body,
        out_shape=jax.ShapeDtypeStruct((N, H), jnp.float32),
    )(x, wcat, t, b.reshape(1, H), wrel)


def _mp_common(hsh, agg, src_v, dst_v, rows, gsem, ssem, KPT, NB, AHEAD):
    for u in range(AHEAD):
        pltpu.async_copy(hsh.at[src_v.at[u]], rows[u], gsem[u])

    def outer(o, carry):
        base = NB * o
        for u in range(NB):
            i = base + u
            v = (u + AHEAD) % NB
            pltpu.make_async_copy(
                hsh.at[src_v.at[i]], rows[u], gsem[u]).wait()
            pltpu.async_copy(
                rows[u], agg.at[dst_v.at[i]], ssem[u], add=True)

            @pl.when(i - AHEAD >= 0)
            def _():
                pltpu.make_async_copy(
                    rows[v], agg.at[dst_v.at[i - AHEAD]], ssem[v]).wait()

            @pl.when(i + AHEAD < KPT)
            def _():
                pltpu.async_copy(
                    hsh.at[src_v.at[i + AHEAD]], rows[v], gsem[v])
        return carry

    lax.fori_loop(0, KPT // NB, outer, 0)
    for k in range(AHEAD):
        i = KPT - AHEAD + k
        u = i % NB
        pltpu.make_async_copy(
            rows[u], agg.at[dst_v.at[i]], ssem[u]).wait()


def _make_mp_round1(N, H, KPT, NPAD):
    mesh = plsc.VectorSubcoreMesh(core_axis_name="c", subcore_axis_name="s")
    rpt = NPAD // _NS

    NB = 8
    AHEAD = 4
    assert KPT % NB == 0

    @functools.partial(
        pl.kernel,
        out_type=jax.ShapeDtypeStruct((_NC * NPAD, H), jnp.float32),
        mesh=mesh,
        scratch_types=(
            [pltpu.VMEM((KPT, _CH), jnp.int32)] * 2
            + [pltpu.VMEM((_CH, H), jnp.float32)] * NB
            + [pltpu.VMEM_SHARED((NPAD, H), jnp.float32)]
            + [pltpu.VMEM_SHARED((NPAD, H), jnp.float32)]
            + [pltpu.SemaphoreType.DMA] * (2 * NB)
        ),
        compiler_params=pltpu.CompilerParams(use_tc_tiling_on_sc=False),
    )
    def mp(h_hbm, src_hbm, dst_hbm, zero_hbm, out_hbm, src_v, dst_v, *rest):
        rows = rest[:NB]
        agg = rest[NB]
        hsh = rest[NB + 1]
        gsem = rest[NB + 2:2 * NB + 2]
        ssem = rest[2 * NB + 2:]
        c = lax.axis_index("c")
        s = lax.axis_index("s")
        wid = c * _NS + s
        pltpu.sync_copy(h_hbm.at[pl.ds(s * rpt, rpt)],
                        hsh.at[pl.ds(s * rpt, rpt)])
        pltpu.sync_copy(zero_hbm.at[pl.ds(s * rpt, rpt)],
                        agg.at[pl.ds(s * rpt, rpt)])
        pltpu.sync_copy(src_hbm.at[pl.ds(wid * KPT, KPT)], src_v)
        pltpu.sync_copy(dst_hbm.at[pl.ds(wid * KPT, KPT)], dst_v)
        plsc.subcore_barrier()

        _mp_common(hsh, agg, src_v, dst_v, rows, gsem, ssem, KPT, NB, AHEAD)

        plsc.subcore_barrier()
        pltpu.sync_copy(agg.at[pl.ds(s * rpt, rpt)],
                        out_hbm.at[pl.ds(c * NPAD + s * rpt, rpt)])

    return mp


def _make_mp_round2(N, H, KPT, NPAD):
    mesh = plsc.VectorSubcoreMesh(core_axis_name="c", subcore_axis_name="s")
    rpt = NPAD // _NS

    NB = 8
    AHEAD = 4
    assert KPT % NB == 0

    @functools.partial(
        pl.kernel,
        out_type=jax.ShapeDtypeStruct((_NC * NPAD, H), jnp.float32),
        mesh=mesh,
        scratch_types=(
            [pltpu.VMEM((KPT, _CH), jnp.int32)] * 2
            + [pltpu.VMEM((_CH, H), jnp.float32)] * NB
            + [pltpu.VMEM((NPAD // _NS, H), jnp.float32)] * 2
            + [pltpu.VMEM_SHARED((NPAD, H), jnp.float32)]
            + [pltpu.VMEM_SHARED((NPAD, H), jnp.float32)]
            + [pltpu.SemaphoreType.DMA] * (2 * NB)
        ),
        compiler_params=pltpu.CompilerParams(use_tc_tiling_on_sc=False),
    )
    def mp(p_hbm, src_hbm, dst_hbm, zero_hbm, out_hbm, src_v, dst_v, *rest):
        rows = rest[:NB]
        a_v = rest[NB]
        b_v = rest[NB + 1]
        agg = rest[NB + 2]
        hsh = rest[NB + 3]
        gsem = rest[NB + 4:2 * NB + 4]
        ssem = rest[2 * NB + 4:]
        c = lax.axis_index("c")
        s = lax.axis_index("s")
        wid = c * _NS + s
        pltpu.sync_copy(p_hbm.at[pl.ds(s * rpt, rpt)], a_v)
        pltpu.sync_copy(p_hbm.at[pl.ds(NPAD + s * rpt, rpt)], b_v)

        def mid(r, carry):
            v = a_v[r] + b_v[r]
            a_v[r] = jnp.where(v >= 0, v, 0.01 * v)
            return carry

        lax.fori_loop(0, rpt, mid, 0)
        pltpu.sync_copy(a_v, hsh.at[pl.ds(s * rpt, rpt)])
        pltpu.sync_copy(zero_hbm.at[pl.ds(s * rpt, rpt)],
                        agg.at[pl.ds(s * rpt, rpt)])
        pltpu.sync_copy(src_hbm.at[pl.ds(wid * KPT, KPT)], src_v)
        pltpu.sync_copy(dst_hbm.at[pl.ds(wid * KPT, KPT)], dst_v)
        plsc.subcore_barrier()

        _mp_common(hsh, agg, src_v, dst_v, rows, gsem, ssem, KPT, NB, AHEAD)

        plsc.subcore_barrier()
        pltpu.sync_copy(agg.at[pl.ds(s * rpt, rpt)],
                        out_hbm.at[pl.ds(c * NPAD + s * rpt, rpt)])

    return mp


def _head(q, wh, bh, wo, bo, wlog_t, blog, N, NPAD, H, OUT):
    def body(q_ref, wh_ref, bh_ref, wo_ref, bo_ref, wl_ref, bl_ref,
             out_ref, emb_ref):
        v = q_ref[:NPAD, :] + q_ref[NPAD:, :]
        h2 = _leaky(
            jnp.dot(v, wh_ref[...], preferred_element_type=jnp.float32)
            + bh_ref[...])
        hn = jnp.dot(h2[:N, :], wo_ref[...],
                     preferred_element_type=jnp.float32)
        g = jnp.sum(hn, axis=0, keepdims=True) / N + bo_ref[...]
        emb = _leaky(g)
        logit = jnp.sum(emb * wl_ref[...], axis=1, keepdims=True) + bl_ref[...]
        out_ref[...] = jax.nn.sigmoid(logit)
        emb_ref[...] = emb

    return pl.pallas_call(
        body,
        out_shape=(jax.ShapeDtypeStruct((1, 1), jnp.float32),
                   jax.ShapeDtypeStruct((1, OUT), jnp.float32)),
    )(q, wh, bh.reshape(1, H), wo, bo.reshape(1, OUT),
      wlog_t, blog.reshape(1, 1))


def kernel(x, edge_index, node_types, edge_types, W_in, b_in, W_rel,
           W_hid, b_hid, W_out, b_out, W_log, b_log):
    N, D = x.shape
    NT, _, H = W_in.shape
    OUT = W_out.shape[1]
    E = edge_index.shape[1]

    wcat = jnp.transpose(W_in, (1, 0, 2)).reshape(D, NT * H)
    hr = _input_transform(x, wcat, node_types.reshape(N, 1), b_in,
                          W_rel[0], NT, H)

    nchunk = -(-E // _CH)
    kpt = -(-nchunk // _NTILE)
    kpt = -(-kpt // 8) * 8
    epad = kpt * _NTILE * _CH
    src = jnp.concatenate(
        [edge_index[0], jnp.zeros((epad - E,), jnp.int32)]).reshape(-1, _CH)
    dst = jnp.concatenate(
        [edge_index[1], jnp.full((epad - E,), N, jnp.int32)]).reshape(-1, _CH)
    npad = -(-N // (_NS * 8)) * (_NS * 8)
    zeros = jnp.zeros((npad, H), jnp.float32)
    hr = jnp.concatenate([hr, jnp.zeros((npad - N, H), jnp.float32)])

    p = _make_mp_round1(N, H, kpt, npad)(hr, src, dst, zeros)
    q = _make_mp_round2(N, H, kpt, npad)(p, src, dst, zeros)
    out, emb = _head(q, W_hid, b_hid, W_out, b_out,
                     jnp.transpose(W_log), b_log, N, npad, H, OUT)
    return out, emb.reshape(OUT)


# speedup vs baseline: 1.6937x; 1.0090x over previous
import functools

import jax
import jax.numpy as jnp
from jax import lax
from jax.experimental import pallas as pl
from jax.experimental.pallas import tpu as pltpu
from jax.experimental.pallas import tpu_sc as plsc

_CH = 128
_NC = 2
_NS = 16
_NTILE = _NC * _NS


def _leaky(v):
    return jnp.where(v >= 0, v, 0.01 * v)


def _input_transform(x, wcat, t, b, wrel, NT, H, NPAD):
    N, D = x.shape
    YW = NT * H

    def body(x_ref, w_ref, t_ref, b_ref, r_ref, o_ref):
        y = jnp.dot(x_ref[...], w_ref[...], preferred_element_type=jnp.float32)
        cc = lax.broadcasted_iota(jnp.int32, (N, YW), 1) // H
        sel = jnp.where(cc == t_ref[...], y, 0.0)
        ri = lax.broadcasted_iota(jnp.int32, (YW, H), 0) % H
        ci = lax.broadcasted_iota(jnp.int32, (YW, H), 1)
        S = (ri == ci).astype(jnp.float32)
        h = jnp.dot(sel, S, preferred_element_type=jnp.float32)
        hr = _leaky(h + b_ref[...])
        o_ref[:N, :] = jnp.dot(hr, r_ref[...],
                               preferred_element_type=jnp.float32)
        o_ref[N:, :] = jnp.zeros((NPAD - N, H), jnp.float32)

    return pl.pallas_call(
        body,
        out_shape=jax.ShapeDtypeStruct((NPAD, H), jnp.float32),
    )(x, wcat, t, b.reshape(1, H), wrel)


def _mp_common(hsh, agg, src_v, dst_v, rows, gsem, ssem, KPT, NB, AHEAD):
    for u in range(AHEAD):
        pltpu.async_copy(hsh.at[src_v.at[u]], rows[u], gsem[u])

    def outer(o, carry):
        base = NB * o
        for u in range(NB):
            i = base + u
            v = (u + AHEAD) % NB
            pltpu.make_async_copy(
                hsh.at[src_v.at[i]], rows[u], gsem[u]).wait()
            pltpu.async_copy(
                rows[u], agg.at[dst_v.at[i]], ssem[u], add=True)

            @pl.when(i - AHEAD >= 0)
            def _():
                pltpu.make_async_copy(
                    rows[v], agg.at[dst_v.at[i - AHEAD]], ssem[v]).wait()

            @pl.when(i + AHEAD < KPT)
            def _():
                pltpu.async_copy(
                    hsh.at[src_v.at[i + AHEAD]], rows[v], gsem[v])
        return carry

    lax.fori_loop(0, KPT // NB, outer, 0)
    for k in range(AHEAD):
        i = KPT - AHEAD + k
        u = i % NB
        pltpu.make_async_copy(
            rows[u], agg.at[dst_v.at[i]], ssem[u]).wait()


def _make_mp_round1(N, H, KPT, NPAD):
    mesh = plsc.VectorSubcoreMesh(core_axis_name="c", subcore_axis_name="s")
    rpt = NPAD // _NS

    NB = 8
    AHEAD = 4
    assert KPT % NB == 0

    @functools.partial(
        pl.kernel,
        out_type=jax.ShapeDtypeStruct((_NC * NPAD, H), jnp.float32),
        mesh=mesh,
        scratch_types=(
            [pltpu.VMEM((KPT, _CH), jnp.int32)] * 2
            + [pltpu.VMEM((_CH, H), jnp.float32)] * NB
            + [pltpu.VMEM_SHARED((NPAD, H), jnp.float32)]
            + [pltpu.VMEM_SHARED((NPAD, H), jnp.float32)]
            + [pltpu.SemaphoreType.DMA] * (2 * NB)
        ),
        compiler_params=pltpu.CompilerParams(use_tc_tiling_on_sc=False),
    )
    def mp(h_hbm, src_hbm, dst_hbm, zero_hbm, out_hbm, src_v, dst_v, *rest):
        rows = rest[:NB]
        agg = rest[NB]
        hsh = rest[NB + 1]
        gsem = rest[NB + 2:2 * NB + 2]
        ssem = rest[2 * NB + 2:]
        c = lax.axis_index("c")
        s = lax.axis_index("s")
        wid = c * _NS + s
        pltpu.sync_copy(h_hbm.at[pl.ds(s * rpt, rpt)],
                        hsh.at[pl.ds(s * rpt, rpt)])
        pltpu.sync_copy(zero_hbm.at[pl.ds(s * rpt, rpt)],
                        agg.at[pl.ds(s * rpt, rpt)])
        pltpu.sync_copy(src_hbm.at[pl.ds(wid * KPT, KPT)], src_v)
        pltpu.sync_copy(dst_hbm.at[pl.ds(wid * KPT, KPT)], dst_v)
        plsc.subcore_barrier()

        _mp_common(hsh, agg, src_v, dst_v, rows, gsem, ssem, KPT, NB, AHEAD)

        plsc.subcore_barrier()
        pltpu.sync_copy(agg.at[pl.ds(s * rpt, rpt)],
                        out_hbm.at[pl.ds(c * NPAD + s * rpt, rpt)])

    return mp


def _make_mp_round2(N, H, KPT, NPAD):
    mesh = plsc.VectorSubcoreMesh(core_axis_name="c", subcore_axis_name="s")
    rpt = NPAD // _NS

    NB = 8
    AHEAD = 4
    assert KPT % NB == 0

    @functools.partial(
        pl.kernel,
        out_type=jax.ShapeDtypeStruct((_NC * NPAD, H), jnp.float32),
        mesh=mesh,
        scratch_types=(
            [pltpu.VMEM((KPT, _CH), jnp.int32)] * 2
            + [pltpu.VMEM((_CH, H), jnp.float32)] * NB
            + [pltpu.VMEM((NPAD // _NS, H), jnp.float32)] * 2
            + [pltpu.VMEM_SHARED((NPAD, H), jnp.float32)]
            + [pltpu.VMEM_SHARED((NPAD, H), jnp.float32)]
            + [pltpu.SemaphoreType.DMA] * (2 * NB)
        ),
        compiler_params=pltpu.CompilerParams(use_tc_tiling_on_sc=False),
    )
    def mp(p_hbm, src_hbm, dst_hbm, zero_hbm, out_hbm, src_v, dst_v, *rest):
        rows = rest[:NB]
        a_v = rest[NB]
        b_v = rest[NB + 1]
        agg = rest[NB + 2]
        hsh = rest[NB + 3]
        gsem = rest[NB + 4:2 * NB + 4]
        ssem = rest[2 * NB + 4:]
        c = lax.axis_index("c")
        s = lax.axis_index("s")
        wid = c * _NS + s
        pltpu.sync_copy(p_hbm.at[pl.ds(s * rpt, rpt)], a_v)
        pltpu.sync_copy(p_hbm.at[pl.ds(NPAD + s * rpt, rpt)], b_v)

        def mid(r, carry):
            v = a_v[r] + b_v[r]
            a_v[r] = jnp.where(v >= 0, v, 0.01 * v)
            return carry

        lax.fori_loop(0, rpt, mid, 0)
        pltpu.sync_copy(a_v, hsh.at[pl.ds(s * rpt, rpt)])
        pltpu.sync_copy(zero_hbm.at[pl.ds(s * rpt, rpt)],
                        agg.at[pl.ds(s * rpt, rpt)])
        pltpu.sync_copy(src_hbm.at[pl.ds(wid * KPT, KPT)], src_v)
        pltpu.sync_copy(dst_hbm.at[pl.ds(wid * KPT, KPT)], dst_v)
        plsc.subcore_barrier()

        _mp_common(hsh, agg, src_v, dst_v, rows, gsem, ssem, KPT, NB, AHEAD)

        plsc.subcore_barrier()
        pltpu.sync_copy(agg.at[pl.ds(s * rpt, rpt)],
                        out_hbm.at[pl.ds(c * NPAD + s * rpt, rpt)])

    return mp


def _head(q, wh, bh, wo, bo, wlog_t, blog, N, NPAD, H, OUT):
    def body(q_ref, wh_ref, bh_ref, wo_ref, bo_ref, wl_ref, bl_ref,
             out_ref, emb_ref):
        v = q_ref[:NPAD, :] + q_ref[NPAD:, :]
        h2 = _leaky(
            jnp.dot(v, wh_ref[...], preferred_element_type=jnp.float32)
            + bh_ref[...])
        hn = jnp.dot(h2[:N, :], wo_ref[...],
                     preferred_element_type=jnp.float32)
        g = jnp.sum(hn, axis=0, keepdims=True) / N + bo_ref[...]
        emb = _leaky(g)
        logit = jnp.sum(emb * wl_ref[...], axis=1, keepdims=True) + bl_ref[...]
        out_ref[...] = jax.nn.sigmoid(logit)
        emb_ref[...] = emb

    return pl.pallas_call(
        body,
        out_shape=(jax.ShapeDtypeStruct((1, 1), jnp.float32),
                   jax.ShapeDtypeStruct((1, OUT), jnp.float32)),
    )(q, wh, bh.reshape(1, H), wo, bo.reshape(1, OUT),
      wlog_t, blog.reshape(1, 1))


def kernel(x, edge_index, node_types, edge_types, W_in, b_in, W_rel,
           W_hid, b_hid, W_out, b_out, W_log, b_log):
    N, D = x.shape
    NT, _, H = W_in.shape
    OUT = W_out.shape[1]
    E = edge_index.shape[1]

    npad = -(-N // (_NS * 8)) * (_NS * 8)
    wcat = jnp.transpose(W_in, (1, 0, 2)).reshape(D, NT * H)
    hr = _input_transform(x, wcat, node_types.reshape(N, 1), b_in,
                          W_rel[0], NT, H, npad)

    nchunk = -(-E // _CH)
    kpt = -(-nchunk // _NTILE)
    kpt = -(-kpt // 8) * 8
    epad = kpt * _NTILE * _CH
    src = jnp.concatenate(
        [edge_index[0], jnp.zeros((epad - E,), jnp.int32)]).reshape(-1, _CH)
    dst = jnp.concatenate(
        [edge_index[1], jnp.full((epad - E,), N, jnp.int32)]).reshape(-1, _CH)
    zeros = jnp.zeros((npad, H), jnp.float32)

    p = _make_mp_round1(N, H, kpt, npad)(hr, src, dst, zeros)
    q = _make_mp_round2(N, H, kpt, npad)(p, src, dst, zeros)
    out, emb = _head(q, W_hid, b_hid, W_out, b_out,
                     jnp.transpose(W_log), b_log, N, npad, H, OUT)
    return out, emb.reshape(OUT)


# speedup vs baseline: 1.7507x; 1.0337x over previous
import functools

import jax
import jax.numpy as jnp
from jax import lax
from jax.experimental import pallas as pl
from jax.experimental.pallas import tpu as pltpu
from jax.experimental.pallas import tpu_sc as plsc

_CH = 128
_NC = 2
_NS = 16
_NTILE = _NC * _NS


def _leaky(v):
    return jnp.where(v >= 0, v, 0.01 * v)


def _input_transform(x, wcat, t, b, wrel, NT, H, NPAD):
    N, D = x.shape
    YW = NT * H

    def body(x_ref, w_ref, t_ref, b_ref, r_ref, o_ref):
        y = jnp.dot(x_ref[...], w_ref[...], preferred_element_type=jnp.float32)
        cc = lax.broadcasted_iota(jnp.int32, (N, YW), 1) // H
        sel = jnp.where(cc == t_ref[...], y, 0.0)
        ri = lax.broadcasted_iota(jnp.int32, (YW, H), 0) % H
        ci = lax.broadcasted_iota(jnp.int32, (YW, H), 1)
        S = (ri == ci).astype(jnp.float32)
        h = jnp.dot(sel, S, preferred_element_type=jnp.float32)
        hr = _leaky(h + b_ref[...])
        o_ref[:N, :] = jnp.dot(hr, r_ref[...],
                               preferred_element_type=jnp.float32)
        o_ref[N:, :] = jnp.zeros((NPAD - N, H), jnp.float32)

    return pl.pallas_call(
        body,
        out_shape=jax.ShapeDtypeStruct((NPAD, H), jnp.float32),
    )(x, wcat, t, b.reshape(1, H), wrel)


def _mp_common(hsh, agg, src_v, dst_v, rows, gsem, ssem, KPT, NB, AHEAD):
    for u in range(AHEAD):
        pltpu.async_copy(hsh.at[src_v.at[u]], rows[u], gsem[u])

    def outer(o, carry):
        base = NB * o
        for u in range(NB):
            i = base + u
            v = (u + AHEAD) % NB
            pltpu.make_async_copy(
                hsh.at[src_v.at[i]], rows[u], gsem[u]).wait()
            pltpu.async_copy(
                rows[u], agg.at[dst_v.at[i]], ssem[u], add=True)

            @pl.when(i - AHEAD >= 0)
            def _():
                pltpu.make_async_copy(
                    rows[v], agg.at[dst_v.at[i - AHEAD]], ssem[v]).wait()

            @pl.when(i + AHEAD < KPT)
            def _():
                pltpu.async_copy(
                    hsh.at[src_v.at[i + AHEAD]], rows[v], gsem[v])
        return carry

    lax.fori_loop(0, KPT // NB, outer, 0)
    for k in range(AHEAD):
        i = KPT - AHEAD + k
        u = i % NB
        pltpu.make_async_copy(
            rows[u], agg.at[dst_v.at[i]], ssem[u]).wait()


def _make_mp_round1(N, H, KPT, NPAD):
    mesh = plsc.VectorSubcoreMesh(core_axis_name="c", subcore_axis_name="s")
    rpt = NPAD // _NS

    NB = 8
    AHEAD = 4
    assert KPT % NB == 0

    @functools.partial(
        pl.kernel,
        out_type=jax.ShapeDtypeStruct((_NC * NPAD, H), jnp.float32),
        mesh=mesh,
        scratch_types=(
            [pltpu.VMEM((KPT, _CH), jnp.int32)] * 2
            + [pltpu.VMEM((_CH, H), jnp.float32)] * NB
            + [pltpu.VMEM_SHARED((NPAD, H), jnp.float32)]
            + [pltpu.VMEM_SHARED((NPAD, H), jnp.float32)]
            + [pltpu.SemaphoreType.DMA] * (2 * NB + 4)
        ),
        compiler_params=pltpu.CompilerParams(use_tc_tiling_on_sc=False),
    )
    def mp(h_hbm, src_hbm, dst_hbm, zero_hbm, out_hbm, src_v, dst_v, *rest):
        rows = rest[:NB]
        agg = rest[NB]
        hsh = rest[NB + 1]
        gsem = rest[NB + 2:2 * NB + 2]
        ssem = rest[2 * NB + 2:3 * NB + 2]
        psem = rest[3 * NB + 2:]
        c = lax.axis_index("c")
        s = lax.axis_index("s")
        wid = c * _NS + s
        pltpu.async_copy(h_hbm.at[pl.ds(s * rpt, rpt)],
                         hsh.at[pl.ds(s * rpt, rpt)], psem[0])
        pltpu.async_copy(zero_hbm.at[pl.ds(s * rpt, rpt)],
                         agg.at[pl.ds(s * rpt, rpt)], psem[1])
        pltpu.async_copy(src_hbm.at[pl.ds(wid * KPT, KPT)], src_v, psem[2])
        pltpu.async_copy(dst_hbm.at[pl.ds(wid * KPT, KPT)], dst_v, psem[3])
        pltpu.make_async_copy(h_hbm.at[pl.ds(s * rpt, rpt)],
                              hsh.at[pl.ds(s * rpt, rpt)], psem[0]).wait()
        pltpu.make_async_copy(zero_hbm.at[pl.ds(s * rpt, rpt)],
                              agg.at[pl.ds(s * rpt, rpt)], psem[1]).wait()
        pltpu.make_async_copy(src_hbm.at[pl.ds(wid * KPT, KPT)],
                              src_v, psem[2]).wait()
        pltpu.make_async_copy(dst_hbm.at[pl.ds(wid * KPT, KPT)],
                              dst_v, psem[3]).wait()
        plsc.subcore_barrier()

        _mp_common(hsh, agg, src_v, dst_v, rows, gsem, ssem, KPT, NB, AHEAD)

        plsc.subcore_barrier()
        pltpu.sync_copy(agg.at[pl.ds(s * rpt, rpt)],
                        out_hbm.at[pl.ds(c * NPAD + s * rpt, rpt)])

    return mp


def _make_mp_round2(N, H, KPT, NPAD):
    mesh = plsc.VectorSubcoreMesh(core_axis_name="c", subcore_axis_name="s")
    rpt = NPAD // _NS

    NB = 8
    AHEAD = 4
    assert KPT % NB == 0

    @functools.partial(
        pl.kernel,
        out_type=jax.ShapeDtypeStruct((_NC * NPAD, H), jnp.float32),
        mesh=mesh,
        scratch_types=(
            [pltpu.VMEM((KPT, _CH), jnp.int32)] * 2
            + [pltpu.VMEM((_CH, H), jnp.float32)] * NB
            + [pltpu.VMEM((NPAD // _NS, H), jnp.float32)] * 2
            + [pltpu.VMEM_SHARED((NPAD, H), jnp.float32)]
            + [pltpu.VMEM_SHARED((NPAD, H), jnp.float32)]
            + [pltpu.SemaphoreType.DMA] * (2 * NB + 5)
        ),
        compiler_params=pltpu.CompilerParams(use_tc_tiling_on_sc=False),
    )
    def mp(p_hbm, src_hbm, dst_hbm, zero_hbm, out_hbm, src_v, dst_v, *rest):
        rows = rest[:NB]
        a_v = rest[NB]
        b_v = rest[NB + 1]
        agg = rest[NB + 2]
        hsh = rest[NB + 3]
        gsem = rest[NB + 4:2 * NB + 4]
        ssem = rest[2 * NB + 4:3 * NB + 4]
        psem = rest[3 * NB + 4:]
        c = lax.axis_index("c")
        s = lax.axis_index("s")
        wid = c * _NS + s
        pltpu.async_copy(p_hbm.at[pl.ds(s * rpt, rpt)], a_v, psem[0])
        pltpu.async_copy(p_hbm.at[pl.ds(NPAD + s * rpt, rpt)], b_v, psem[1])
        pltpu.async_copy(zero_hbm.at[pl.ds(s * rpt, rpt)],
                         agg.at[pl.ds(s * rpt, rpt)], psem[2])
        pltpu.async_copy(src_hbm.at[pl.ds(wid * KPT, KPT)], src_v, psem[3])
        pltpu.async_copy(dst_hbm.at[pl.ds(wid * KPT, KPT)], dst_v, psem[4])
        pltpu.make_async_copy(p_hbm.at[pl.ds(s * rpt, rpt)],
                              a_v, psem[0]).wait()
        pltpu.make_async_copy(p_hbm.at[pl.ds(NPAD + s * rpt, rpt)],
                              b_v, psem[1]).wait()

        def mid(r, carry):
            v = a_v[r] + b_v[r]
            a_v[r] = jnp.where(v >= 0, v, 0.01 * v)
            return carry

        lax.fori_loop(0, rpt, mid, 0)
        pltpu.sync_copy(a_v, hsh.at[pl.ds(s * rpt, rpt)])
        pltpu.make_async_copy(zero_hbm.at[pl.ds(s * rpt, rpt)],
                              agg.at[pl.ds(s * rpt, rpt)], psem[2]).wait()
        pltpu.make_async_copy(src_hbm.at[pl.ds(wid * KPT, KPT)],
                              src_v, psem[3]).wait()
        pltpu.make_async_copy(dst_hbm.at[pl.ds(wid * KPT, KPT)],
                              dst_v, psem[4]).wait()
        plsc.subcore_barrier()

        _mp_common(hsh, agg, src_v, dst_v, rows, gsem, ssem, KPT, NB, AHEAD)

        plsc.subcore_barrier()
        pltpu.sync_copy(agg.at[pl.ds(s * rpt, rpt)],
                        out_hbm.at[pl.ds(c * NPAD + s * rpt, rpt)])

    return mp


def _head(q, wh, bh, wo, bo, wlog_t, blog, N, NPAD, H, OUT):
    def body(q_ref, wh_ref, bh_ref, wo_ref, bo_ref, wl_ref, bl_ref,
             out_ref, emb_ref):
        v = q_ref[:NPAD, :] + q_ref[NPAD:, :]
        h2 = _leaky(
            jnp.dot(v, wh_ref[...], preferred_element_type=jnp.float32)
            + bh_ref[...])
        hn = jnp.dot(h2[:N, :], wo_ref[...],
                     preferred_element_type=jnp.float32)
        g = jnp.sum(hn, axis=0, keepdims=True) / N + bo_ref[...]
        emb = _leaky(g)
        logit = jnp.sum(emb * wl_ref[...], axis=1, keepdims=True) + bl_ref[...]
        out_ref[...] = jax.nn.sigmoid(logit)
        emb_ref[...] = emb

    return pl.pallas_call(
        body,
        out_shape=(jax.ShapeDtypeStruct((1, 1), jnp.float32),
                   jax.ShapeDtypeStruct((1, OUT), jnp.float32)),
    )(q, wh, bh.reshape(1, H), wo, bo.reshape(1, OUT),
      wlog_t, blog.reshape(1, 1))


def kernel(x, edge_index, node_types, edge_types, W_in, b_in, W_rel,
           W_hid, b_hid, W_out, b_out, W_log, b_log):
    N, D = x.shape
    NT, _, H = W_in.shape
    OUT = W_out.shape[1]
    E = edge_index.shape[1]

    npad = -(-N // (_NS * 8)) * (_NS * 8)
    wcat = jnp.transpose(W_in, (1, 0, 2)).reshape(D, NT * H)
    hr = _input_transform(x, wcat, node_types.reshape(N, 1), b_in,
                          W_rel[0], NT, H, npad)

    nchunk = -(-E // _CH)
    kpt = -(-nchunk // _NTILE)
    kpt = -(-kpt // 8) * 8
    epad = kpt * _NTILE * _CH
    src = jnp.concatenate(
        [edge_index[0], jnp.zeros((epad - E,), jnp.int32)]).reshape(-1, _CH)
    dst = jnp.concatenate(
        [edge_index[1], jnp.full((epad - E,), N, jnp.int32)]).reshape(-1, _CH)
    zeros = jnp.zeros((npad, H), jnp.float32)

    p = _make_mp_round1(N, H, kpt, npad)(hr, src, dst, zeros)
    q = _make_mp_round2(N, H, kpt, npad)(p, src, dst, zeros)
    out, emb = _head(q, W_hid, b_hid, W_out, b_out,
                     jnp.transpose(W_log), b_log, N, npad, H, OUT)
    return out, emb.reshape(OUT)
